# baseline (speedup 1.0000x reference)
import jax
import jax.numpy as jnp
from jax import lax
from jax.experimental import pallas as pl
from jax.experimental.pallas import tpu as pltpu
from jax.experimental.pallas import tpu_sc as plsc

_B = 16384
_F = 64
_L = 16
_NC = 2
_NS = 16
_NW = _NC * _NS
_PER_W = _B // _NW
_PH = 128
_NPH = _PER_W // _PH

_mesh = plsc.VectorSubcoreMesh(core_axis_name="c", subcore_axis_name="s")


def _scalar(vec, j):
    return lax.squeeze(lax.slice_in_dim(vec, j, j + 1), (0,))


def _sc_body(x_hbm, lab_hbm, cen_hbm, out_hbm,
             idx_v, x_v, rows0_v, rows1_v, acc_v, sem0, sem1, xsem):
    wid = lax.axis_index("s") * _NC + lax.axis_index("c")
    base = wid * _PER_W
    bufs = (rows0_v, rows1_v)
    sems = (sem0, sem1)

    pltpu.sync_copy(lab_hbm.at[pl.ds(base, _PER_W)], idx_v)

    def fire(p):
        buf, sem = bufs[p % 2], sems[p % 2]

        @pl.loop(0, _PH // _L)
        def _(ch):
            lv = idx_v[pl.ds(p * _PH + ch * _L, _L)]
            for j in range(_L):
                l = _scalar(lv, j)
                t = lax.shift_right_logical(l, 3)
                q = jnp.bitwise_and(l, 7)
                pltpu.async_copy(cen_hbm.at[t, q], buf.at[ch * _L + j], sem)

    fire(0)
    xcopy = pltpu.make_async_copy(
        x_hbm.at[pl.ds(base * _F, _PER_W * _F)], x_v, xsem
    )
    xcopy.start()

    acc = jnp.zeros((_L,), jnp.float32)
    for p in range(_NPH):
        if p + 1 < _NPH:
            fire(p + 1)
        if p == 0:
            xcopy.wait()
        buf, sem = bufs[p % 2], sems[p % 2]

        @pl.loop(0, _PH)
        def _(r):
            pltpu.make_async_copy(cen_hbm.at[0, 0], buf.at[0], sem).wait()

        def pair_body(i, acc):
            for k in range(2):
                r = i * 2 + k
                rb = (p * _PH) * _F + r * _F
                for cc in range(_F // _L):
                    xv = x_v[pl.ds(rb + cc * _L, _L)]
                    cv = buf[r, pl.ds(cc * _L, _L)]
                    d = xv - cv
                    acc = acc + d * d
            return acc

        acc = lax.fori_loop(0, _PH // 2, pair_body, acc)

    acc_v[...] = acc * (0.5 / _B)
    pltpu.sync_copy(acc_v, out_hbm.at[wid])


@jax.jit
def kernel(x, labels, centers):
    labels = labels.astype(jnp.int32)
    x = x.reshape(-1)
    centers = centers.reshape(-1, 8, _F)
    run = pl.kernel(
        _sc_body,
        out_type=jax.ShapeDtypeStruct((_NW, _L), jnp.float32),
        mesh=_mesh,
        compiler_params=pltpu.CompilerParams(needs_layout_passes=False),
        scratch_types=[
            pltpu.VMEM((_PER_W,), jnp.int32),
            pltpu.VMEM((_PER_W * _F,), jnp.float32),
            pltpu.VMEM((_PH, _F), jnp.float32),
            pltpu.VMEM((_PH, _F), jnp.float32),
            pltpu.VMEM((_L,), jnp.float32),
            pltpu.SemaphoreType.DMA,
            pltpu.SemaphoreType.DMA,
            pltpu.SemaphoreType.DMA,
        ],
    )
    partials = run(x, labels, centers)
    return jnp.sum(partials)


# speedup vs baseline: 1.0455x; 1.0455x over previous
import jax
import jax.numpy as jnp
from jax import lax
from jax.experimental import pallas as pl
from jax.experimental.pallas import tpu as pltpu
from jax.experimental.pallas import tpu_sc as plsc

_B = 16384
_F = 64
_L = 16
_NC = 2
_NS = 16
_NW = _NC * _NS
_PER_W = _B // _NW
_PH = 256
_NPH = _PER_W // _PH

_mesh = plsc.VectorSubcoreMesh(core_axis_name="c", subcore_axis_name="s")


def _scalar(vec, j):
    return lax.squeeze(lax.slice_in_dim(vec, j, j + 1), (0,))


def _sc_body(x_hbm, lab_hbm, cen_hbm, out_hbm,
             idx_v, x_v, rows_v, acc_v, sem, xsem):
    wid = lax.axis_index("s") * _NC + lax.axis_index("c")
    base = wid * _PER_W

    pltpu.sync_copy(lab_hbm.at[pl.ds(base, _PER_W)], idx_v)

    acc = jnp.zeros((_L,), jnp.float32)
    for p in range(_NPH):
        @pl.loop(0, _PH // _L)
        def _(ch):
            lv = idx_v[pl.ds(p * _PH + ch * _L, _L)]
            for j in range(_L):
                l = _scalar(lv, j)
                t = lax.shift_right_logical(l, 3)
                q = jnp.bitwise_and(l, 7)
                pltpu.async_copy(
                    cen_hbm.at[t, q], rows_v.at[ch * _L + j], sem
                )

        pltpu.async_copy(
            x_hbm.at[pl.ds(base + p * _PH, _PH)], x_v, xsem
        ).wait()

        @pl.loop(0, _PH)
        def _(r):
            pltpu.make_async_copy(cen_hbm.at[0, 0], rows_v.at[0], sem).wait()

        def row_body(r, acc):
            for cc in range(_F // _L):
                xv = x_v[r, pl.ds(cc * _L, _L)]
                cv = rows_v[r, pl.ds(cc * _L, _L)]
                d = xv - cv
                acc = acc + d * d
            return acc

        acc = lax.fori_loop(0, _PH, row_body, acc)

    acc_v[...] = acc * (0.5 / _B)
    pltpu.sync_copy(acc_v, out_hbm.at[wid])


@jax.jit
def kernel(x, labels, centers):
    labels = labels.astype(jnp.int32)
    centers = centers.reshape(-1, 8, _F)
    run = pl.kernel(
        _sc_body,
        out_type=jax.ShapeDtypeStruct((_NW, _L), jnp.float32),
        mesh=_mesh,
        compiler_params=pltpu.CompilerParams(needs_layout_passes=False),
        scratch_types=[
            pltpu.VMEM((_PER_W,), jnp.int32),
            pltpu.VMEM((_PH, _F), jnp.float32),
            pltpu.VMEM((_PH, _F), jnp.float32),
            pltpu.VMEM((_L,), jnp.float32),
            pltpu.SemaphoreType.DMA,
            pltpu.SemaphoreType.DMA,
        ],
    )
    partials = run(x, labels, centers)
    return jnp.sum(partials)


# speedup vs baseline: 1.0833x; 1.0361x over previous
import jax
import jax.numpy as jnp
from jax import lax
from jax.experimental import pallas as pl
from jax.experimental.pallas import tpu as pltpu
from jax.experimental.pallas import tpu_sc as plsc

_B = 16384
_F = 64
_L = 16
_NC = 2
_NS = 16
_NW = _NC * _NS
_PER_W = _B // _NW
_PH = 128
_NPH = _PER_W // _PH

_mesh = plsc.VectorSubcoreMesh(core_axis_name="c", subcore_axis_name="s")


def _scalar(vec, j):
    return lax.squeeze(lax.slice_in_dim(vec, j, j + 1), (0,))


def _sc_body(x_hbm, lab_hbm, cen_hbm, out_hbm,
             idx_v, x0_v, x1_v, rows0_v, rows1_v, acc_v,
             sem0, sem1, xsem0, xsem1):
    wid = lax.axis_index("s") * _NC + lax.axis_index("c")
    base = wid * _PER_W
    rbufs = (rows0_v, rows1_v)
    xbufs = (x0_v, x1_v)
    sems = (sem0, sem1)
    xsems = (xsem0, xsem1)

    pltpu.sync_copy(lab_hbm.at[pl.ds(base, _PER_W)], idx_v)

    def fire(p):
        buf, sem = rbufs[p % 2], sems[p % 2]

        @pl.loop(0, _PH // _L)
        def _(ch):
            lv = idx_v[pl.ds(p * _PH + ch * _L, _L)]
            for j in range(_L):
                l = _scalar(lv, j)
                t = lax.shift_right_logical(l, 3)
                q = jnp.bitwise_and(l, 7)
                pltpu.async_copy(cen_hbm.at[t, q], buf.at[ch * _L + j], sem)

    def fire_x(p):
        return pltpu.make_async_copy(
            x_hbm.at[pl.ds(base + p * _PH, _PH)], xbufs[p % 2], xsems[p % 2]
        )

    fire(0)
    fire_x(0).start()

    acc = jnp.zeros((_L,), jnp.float32)
    for p in range(_NPH):
        if p + 1 < _NPH:
            fire(p + 1)
            fire_x(p + 1).start()
        rbuf, sem, xbuf = rbufs[p % 2], sems[p % 2], xbufs[p % 2]

        @pl.loop(0, _PH)
        def _(r):
            pltpu.make_async_copy(cen_hbm.at[0, 0], rbuf.at[0], sem).wait()

        fire_x(p).wait()

        def pair_body(i, acc):
            for k in range(2):
                r = i * 2 + k
                for cc in range(_F // _L):
                    xv = xbuf[r, pl.ds(cc * _L, _L)]
                    cv = rbuf[r, pl.ds(cc * _L, _L)]
                    d = xv - cv
                    acc = acc + d * d
            return acc

        acc = lax.fori_loop(0, _PH // 2, pair_body, acc)

    acc_v[...] = acc * (0.5 / _B)
    pltpu.sync_copy(acc_v, out_hbm.at[wid])


@jax.jit
def kernel(x, labels, centers):
    labels = labels.astype(jnp.int32)
    centers = centers.reshape(-1, 8, _F)
    run = pl.kernel(
        _sc_body,
        out_type=jax.ShapeDtypeStruct((_NW, _L), jnp.float32),
        mesh=_mesh,
        compiler_params=pltpu.CompilerParams(needs_layout_passes=False),
        scratch_types=[
            pltpu.VMEM((_PER_W,), jnp.int32),
            pltpu.VMEM((_PH, _F), jnp.float32),
            pltpu.VMEM((_PH, _F), jnp.float32),
            pltpu.VMEM((_PH, _F), jnp.float32),
            pltpu.VMEM((_PH, _F), jnp.float32),
            pltpu.VMEM((_L,), jnp.float32),
            pltpu.SemaphoreType.DMA,
            pltpu.SemaphoreType.DMA,
            pltpu.SemaphoreType.DMA,
            pltpu.SemaphoreType.DMA,
        ],
    )
    partials = run(x, labels, centers)
    return jnp.sum(partials)
